# baseline (device time: 9590 ns/iter reference)
import jax
import jax.numpy as jnp
from jax import lax
from jax.experimental import pallas as pl
from jax.experimental.pallas import tpu as pltpu

N_DEV = 4

N_SEM = 10


def kernel(x):
    m_per, n = x.shape
    m_half = m_per // 2
    m_q = m_per // 4

    def body(x_ref, out_ref, send_sems, recv_sems, copy_sem):
        my_pos = lax.axis_index("i")
        left = (my_pos - 1) % N_DEV
        right = (my_pos + 1) % N_DEV
        opp = (my_pos + 2) % N_DEV

        def rdma(src, dst, slot, target):
            return pltpu.make_async_remote_copy(
                src_ref=src,
                dst_ref=dst,
                send_sem=send_sems.at[slot],
                recv_sem=recv_sems.at[slot],
                device_id=(target,),
                device_id_type=pl.DeviceIdType.MESH,
            )

        def row(base, off, size):
            return pl.ds(base * m_per + off, size)

        local_copy = pltpu.make_async_copy(
            x_ref, out_ref.at[pl.ds(my_pos * m_per, m_per), :], copy_sem
        )
        local_copy.start()

        barrier_sem = pltpu.get_barrier_semaphore()
        for nbr in [left, right]:
            pl.semaphore_signal(
                barrier_sem, inc=1,
                device_id=(nbr,), device_id_type=pl.DeviceIdType.MESH,
            )
        pl.semaphore_wait(barrier_sem, 2)

        sends = []
        for slot, (src_off, size, tgt) in enumerate([
            (0, m_q, right),
            (m_q, m_q, right),
            (m_half, m_half, right),
            (m_half, m_q, left),
            (m_half + m_q, m_q, left),
            (0, m_half, left),
        ]):
            s = rdma(
                x_ref.at[pl.ds(src_off, size), :],
                out_ref.at[row(my_pos, src_off, size), :],
                slot, tgt,
            )
            s.start()
            sends.append(s)

        q_src = x_ref.at[pl.ds(0, m_q), :]
        h_src = x_ref.at[pl.ds(0, m_half), :]

        for k, (rslot, fslot) in enumerate([(0, 6), (1, 7)]):
            sl = row(left, k * m_q, m_q)
            rdma(q_src, out_ref.at[sl, :], rslot, left).wait_recv()
            f = rdma(out_ref.at[sl, :], out_ref.at[sl, :], fslot, right)
            f.start()
            sends.append(f)

        for k, (rslot, fslot) in enumerate([(3, 8), (4, 9)]):
            sl = row(right, m_half + k * m_q, m_q)
            rdma(q_src, out_ref.at[sl, :], rslot, right).wait_recv()
            f = rdma(out_ref.at[sl, :], out_ref.at[sl, :], fslot, left)
            f.start()
            sends.append(f)

        rdma(h_src, out_ref.at[row(left, m_half, m_half), :], 2, left).wait_recv()
        rdma(h_src, out_ref.at[row(right, 0, m_half), :], 5, right).wait_recv()

        rdma(q_src, out_ref.at[row(opp, 0, m_q), :], 6, left).wait_recv()
        rdma(q_src, out_ref.at[row(opp, m_q, m_q), :], 7, left).wait_recv()
        rdma(q_src, out_ref.at[row(opp, m_half, m_q), :], 8, right).wait_recv()
        rdma(q_src, out_ref.at[row(opp, m_half + m_q, m_q), :], 9, right).wait_recv()

        for s in sends:
            s.wait_send()
        local_copy.wait()

    return pl.pallas_call(
        body,
        out_shape=jax.ShapeDtypeStruct((N_DEV * m_per, n), x.dtype),
        in_specs=[pl.BlockSpec(memory_space=pltpu.VMEM)],
        out_specs=pl.BlockSpec(memory_space=pltpu.VMEM),
        scratch_shapes=[
            pltpu.SemaphoreType.DMA((N_SEM,)),
            pltpu.SemaphoreType.DMA((N_SEM,)),
            pltpu.SemaphoreType.DMA,
        ],
        compiler_params=pltpu.CompilerParams(collective_id=0),
    )(x)
